# Initial kernel scaffold; baseline (speedup 1.0000x reference)
import jax
import jax.numpy as jnp
from jax.experimental import pallas as pl


def kernel(x, edge_index, edge_values, W0, W1, W2):
    raise NotImplementedError("write your pallas kernel here")


# speedup vs baseline: 3.8366x; 3.8366x over previous
import functools

import jax
import jax.numpy as jnp
from jax import lax
from jax.experimental import pallas as pl
from jax.experimental.pallas import tpu as pltpu
from jax.experimental.pallas import tpu_sc as plsc

N = 10000
E = 320000
D = 128
LANES = 16
NUM_CORES = 2
NUM_SUBCORES = 16
NW = NUM_CORES * NUM_SUBCORES
CHUNK = 128
EPW = ((E + NW * CHUNK - 1) // (NW * CHUNK)) * CHUNK
EPAD = EPW * NW
NCHUNK = EPW // CHUNK
NPAD = 10240
ROWS_PER_TILE = NPAD // NUM_SUBCORES


def _make_agg():
    mesh = plsc.VectorSubcoreMesh(core_axis_name="c", subcore_axis_name="s")

    @functools.partial(
        pl.kernel,
        mesh=mesh,
        out_type=jax.ShapeDtypeStruct((NUM_CORES, NPAD, D), jnp.float32),
        scratch_types=[
            pltpu.VMEM((CHUNK,), jnp.int32),
            pltpu.VMEM((CHUNK,), jnp.int32),
            pltpu.VMEM((CHUNK,), jnp.float32),
            pltpu.VMEM((CHUNK, D), jnp.float32),
            pltpu.VMEM_SHARED((NPAD, D), jnp.float32),
            pltpu.SemaphoreType.DMA,
        ],
    )
    def agg(x_hbm, src_hbm, dst_hbm, val_hbm, zeros_hbm, out_hbm,
            srci, dsti, valv, rows, acc, sem):
        cid = lax.axis_index("c")
        sid = lax.axis_index("s")
        rbase = sid * ROWS_PER_TILE
        pltpu.sync_copy(zeros_hbm, acc.at[pl.ds(rbase, ROWS_PER_TILE)])
        plsc.subcore_barrier()

        ebase = cid * (EPAD // NUM_CORES) + sid * EPW

        def chunk_body(i, carry):
            off = ebase + i * CHUNK
            pltpu.sync_copy(src_hbm.at[pl.ds(off, CHUNK)], srci)
            pltpu.sync_copy(dst_hbm.at[pl.ds(off, CHUNK)], dsti)
            pltpu.sync_copy(val_hbm.at[pl.ds(off, CHUNK)], valv)
            pltpu.async_copy(x_hbm.at[srci], rows, sem).wait()

            def grp_body(g, c2):
                vv = valv[pl.ds(g * LANES, LANES)]
                for k in range(LANES):
                    vb = jnp.full((LANES,), vv[k], jnp.float32)
                    e = g * LANES + k
                    for j in range(D // LANES):
                        sl = pl.ds(j * LANES, LANES)
                        rows[e, sl] = rows[e, sl] * vb
                return c2

            lax.fori_loop(0, CHUNK // LANES, grp_body, 0)
            pltpu.sync_copy(rows, acc.at[dsti], add=True)
            return carry

        lax.fori_loop(0, NCHUNK, chunk_body, 0)
        plsc.subcore_barrier()
        pltpu.sync_copy(acc.at[pl.ds(rbase, ROWS_PER_TILE)],
                        out_hbm.at[cid, pl.ds(rbase, ROWS_PER_TILE)])

    return agg


_agg = _make_agg()


def _mlp_body(p0_ref, p1_ref, w0_ref, w1_ref, w2_ref, o_ref):
    a = p0_ref[...] + p1_ref[...]
    h = jnp.maximum(jnp.dot(a, w0_ref[...], preferred_element_type=jnp.float32), 0.0)
    h = jnp.maximum(jnp.dot(h, w1_ref[...], preferred_element_type=jnp.float32), 0.0)
    o_ref[...] = jnp.dot(h, w2_ref[...], preferred_element_type=jnp.float32)


def _mlp(p0, p1, w0t, w1t, w2t):
    blk = 640
    grid = NPAD // blk
    wspec = pl.BlockSpec((D, D), lambda i: (0, 0))
    return pl.pallas_call(
        _mlp_body,
        grid=(grid,),
        in_specs=[
            pl.BlockSpec((blk, D), lambda i: (i, 0)),
            pl.BlockSpec((blk, D), lambda i: (i, 0)),
            wspec, wspec, wspec,
        ],
        out_specs=pl.BlockSpec((blk, D), lambda i: (i, 0)),
        out_shape=jax.ShapeDtypeStruct((NPAD, D), jnp.float32),
    )(p0, p1, w0t, w1t, w2t)


def kernel(x, edge_index, edge_values, W0, W1, W2):
    pad = EPAD - E
    dst = jnp.concatenate([edge_index[0], jnp.zeros((pad,), jnp.int32)])
    src = jnp.concatenate([edge_index[1], jnp.zeros((pad,), jnp.int32)])
    val = jnp.concatenate([edge_values, jnp.zeros((pad,), jnp.float32)])
    zeros = jnp.zeros((ROWS_PER_TILE, D), jnp.float32)
    partials = _agg(x, src, dst, val, zeros)
    return _mlp(partials[0], partials[1], W0.T, W1.T, W2.T)[:N]
